# baseline (speedup 1.0000x reference)
import functools

import jax
import jax.numpy as jnp
from jax import lax
from jax.experimental import pallas as pl
from jax.experimental.pallas import tpu as pltpu
from jax.experimental.pallas import tpu_sc as plsc

N_NODES = 10000
D_HID = 128
D_OUT = 40
D_OUTP = 48
NC, NS = 2, 16
NW = NC * NS
K = 128
N_PAD = 10240
RPT = N_PAD // NS
E = 320000
C = 80
E_PAD = NW * C * K
BR = 256
NBUF = 2
BC = 16
NBLK = C // BC


def _sc_mesh():
    return plsc.VectorSubcoreMesh(core_axis_name="c", subcore_axis_name="s")


def _degrees(src2, dst2, zeros1):
    @functools.partial(
        pl.kernel,
        out_type=jax.ShapeDtypeStruct((4 * N_PAD,), jnp.float32),
        mesh=_sc_mesh(),
        scratch_types=[
            pltpu.VMEM((C, K), jnp.int32),
            pltpu.VMEM((C, K), jnp.int32),
            pltpu.VMEM((K,), jnp.float32),
            pltpu.VMEM_SHARED((N_PAD,), jnp.float32),
            pltpu.VMEM_SHARED((N_PAD,), jnp.float32),
        ],
    )
    def k(src_hbm, dst_hbm, z_hbm, out_hbm, idx_s, idx_d, ones_v, acc_o, acc_i):
        c = lax.axis_index("c")
        s = lax.axis_index("s")
        t = c * NS + s
        pltpu.sync_copy(src_hbm.at[pl.ds(t * C, C)], idx_s)
        pltpu.sync_copy(dst_hbm.at[pl.ds(t * C, C)], idx_d)
        for i in range(K // 16):
            ones_v[pl.ds(i * 16, 16)] = jnp.ones((16,), jnp.float32)
        pltpu.sync_copy(z_hbm, acc_o.at[pl.ds(s * RPT, RPT)])
        pltpu.sync_copy(z_hbm, acc_i.at[pl.ds(s * RPT, RPT)])
        plsc.subcore_barrier()

        def body(j, carry):
            pltpu.sync_copy(ones_v, acc_o.at[idx_s.at[j]], add=True)
            pltpu.sync_copy(ones_v, acc_i.at[idx_d.at[j]], add=True)
            return carry

        lax.fori_loop(0, C, body, 0)
        plsc.subcore_barrier()
        pltpu.sync_copy(acc_o.at[pl.ds(s * RPT, RPT)],
                        out_hbm.at[pl.ds(c * N_PAD + s * RPT, RPT)])
        pltpu.sync_copy(acc_i.at[pl.ds(s * RPT, RPT)],
                        out_hbm.at[pl.ds((2 + c) * N_PAD + s * RPT, RPT)])

    return k(src2, dst2, zeros1)


def _propagate(xs, src2, dst2, zrows, d):
    @functools.partial(
        pl.kernel,
        out_type=jax.ShapeDtypeStruct((2 * N_PAD, d), jnp.float32),
        mesh=_sc_mesh(),
        scratch_types=[
            pltpu.VMEM((2, BC, K), jnp.int32),
            pltpu.VMEM((2, BC, K), jnp.int32),
            pltpu.VMEM((NBUF, K, d), jnp.float32),
            pltpu.SemaphoreType.DMA((NBUF,)),
            pltpu.SemaphoreType.DMA((NBUF,)),
            pltpu.SemaphoreType.DMA((2,)),
            pltpu.VMEM_SHARED((N_PAD, d), jnp.float32),
        ],
    )
    def k(xs_hbm, src_hbm, dst_hbm, z_hbm, out_hbm, sidx, didx, buf, gsem,
          ssem, isem, acc):
        c = lax.axis_index("c")
        s = lax.axis_index("s")
        base = (c * NS + s) * C

        def load_idx_block(blk, slot):
            off = base + blk * BC
            pltpu.async_copy(src_hbm.at[pl.ds(off, BC)], sidx.at[slot],
                             isem.at[slot])
            pltpu.async_copy(dst_hbm.at[pl.ds(off, BC)], didx.at[slot],
                             isem.at[slot])

        def wait_idx_block(slot):
            pltpu.make_async_copy(src_hbm.at[pl.ds(base, BC)], sidx.at[slot],
                                  isem.at[slot]).wait()
            pltpu.make_async_copy(src_hbm.at[pl.ds(base, BC)], didx.at[slot],
                                  isem.at[slot]).wait()

        load_idx_block(0, 0)
        load_idx_block(1, 1)
        pltpu.sync_copy(z_hbm, acc.at[pl.ds(s * RPT, RPT)])
        wait_idx_block(0)
        pltpu.async_copy(xs_hbm.at[sidx.at[0, 0]], buf.at[0], gsem.at[0])
        plsc.subcore_barrier()

        def body(j, carry):
            blk = lax.div(j, BC)

            @pl.when(jnp.logical_and(lax.rem(j, BC) == 0,
                                     jnp.logical_and(blk >= 1,
                                                     blk + 1 < NBLK)))
            def _():
                load_idx_block(blk + 1, lax.rem(blk + 1, 2))

            b = lax.rem(j, NBUF)
            slot = lax.rem(blk, 2)
            pltpu.make_async_copy(xs_hbm.at[pl.ds(0, K)], buf.at[b],
                                  gsem.at[b]).wait()
            pltpu.async_copy(buf.at[b], acc.at[didx.at[slot, lax.rem(j, BC)]],
                             ssem.at[b], add=True)

            jn = j + 1
            slot_n = lax.rem(lax.div(jn, BC), 2)

            @pl.when(jn < C)
            def _():
                @pl.when(lax.rem(jn, BC) == 0)
                def _():
                    wait_idx_block(slot_n)

                bn = lax.rem(jn, NBUF)

                @pl.when(jn >= NBUF)
                def _():
                    pltpu.make_async_copy(buf.at[bn],
                                          acc.at[didx.at[0, 0]],
                                          ssem.at[bn]).wait()

                pltpu.async_copy(xs_hbm.at[sidx.at[slot_n, lax.rem(jn, BC)]],
                                 buf.at[bn], gsem.at[bn])

            return carry

        lax.fori_loop(0, C, body, 0)
        pltpu.make_async_copy(buf.at[(C - 1) % NBUF], acc.at[didx.at[0, 0]],
                              ssem.at[(C - 1) % NBUF]).wait()
        plsc.subcore_barrier()
        pltpu.sync_copy(acc.at[pl.ds(s * RPT, RPT)],
                        out_hbm.at[pl.ds(c * N_PAD + s * RPT, RPT)])

    return k(xs, src2, dst2, zrows)


def _norms(dblk):
    ns = lax.rsqrt(jnp.maximum(dblk[:, 0] + dblk[:, 1], 1.0))
    nd = lax.rsqrt(jnp.maximum(dblk[:, 2] + dblk[:, 3], 1.0))
    return ns, nd


def _tc_prelude(feats_p, degs_t):
    def body(f_ref, d_ref, o_ref):
        ns, _ = _norms(d_ref[...])
        o_ref[...] = f_ref[...] * ns[:, None]

    return pl.pallas_call(
        body,
        grid=(N_PAD // BR,),
        in_specs=[pl.BlockSpec((BR, D_HID), lambda i: (i, 0)),
                  pl.BlockSpec((BR, 4), lambda i: (i, 0))],
        out_specs=pl.BlockSpec((BR, D_HID), lambda i: (i, 0)),
        out_shape=jax.ShapeDtypeStruct((N_PAD, D_HID), jnp.float32),
    )(feats_p, degs_t)


def _tc_layer(parts, degs_t, W, b2d):
    def body(a0, a1, d_ref, w_ref, b_ref, o_ref):
        ns, nd = _norms(d_ref[...])
        agg = (a0[...] + a1[...]) * nd[:, None]
        h = jnp.dot(agg, w_ref[...], preferred_element_type=jnp.float32)
        h = jnp.maximum(h + b_ref[...], 0.0)
        o_ref[...] = h * ns[:, None]

    nblk = N_PAD // BR
    return pl.pallas_call(
        body,
        grid=(nblk,),
        in_specs=[pl.BlockSpec((BR, D_HID), lambda i: (i, 0)),
                  pl.BlockSpec((BR, D_HID), lambda i: (i + nblk, 0)),
                  pl.BlockSpec((BR, 4), lambda i: (i, 0)),
                  pl.BlockSpec((D_HID, D_HID), lambda i: (0, 0)),
                  pl.BlockSpec((1, D_HID), lambda i: (0, 0))],
        out_specs=pl.BlockSpec((BR, D_HID), lambda i: (i, 0)),
        out_shape=jax.ShapeDtypeStruct((N_PAD, D_HID), jnp.float32),
    )(parts, parts, degs_t, W, b2d)


def _tc_layer_prior(parts, degs_t, W, b2d):
    def body(a0, a1, d_ref, w_ref, b_ref, prior_ref, xs_ref):
        ns, nd = _norms(d_ref[...])
        agg = (a0[...] + a1[...]) * nd[:, None]
        h = jnp.dot(agg, w_ref[...], preferred_element_type=jnp.float32)
        h = jnp.maximum(h + b_ref[...], 0.0)
        prior_ref[...] = h
        xs_ref[...] = h * ns[:, None]

    nblk = N_PAD // BR
    return pl.pallas_call(
        body,
        grid=(nblk,),
        in_specs=[pl.BlockSpec((BR, D_HID), lambda i: (i, 0)),
                  pl.BlockSpec((BR, D_HID), lambda i: (i + nblk, 0)),
                  pl.BlockSpec((BR, 4), lambda i: (i, 0)),
                  pl.BlockSpec((D_HID, D_HID), lambda i: (0, 0)),
                  pl.BlockSpec((1, D_HID), lambda i: (0, 0))],
        out_specs=[pl.BlockSpec((BR, D_HID), lambda i: (i, 0)),
                   pl.BlockSpec((BR, D_HID), lambda i: (i, 0))],
        out_shape=[jax.ShapeDtypeStruct((N_PAD, D_HID), jnp.float32),
                   jax.ShapeDtypeStruct((N_PAD, D_HID), jnp.float32)],
    )(parts, parts, degs_t, W, b2d)


def _tc_final(parts, degs_t, W2p, b2d):
    def body(a0, a1, d_ref, w2_ref, b_ref, o_ref):
        _, nd = _norms(d_ref[...])
        agg = (a0[...] + a1[...]) * nd[:, None]
        h = jnp.dot(agg, w2_ref[...], preferred_element_type=jnp.float32)
        o_ref[...] = h + b_ref[...]

    nblk = N_PAD // BR
    return pl.pallas_call(
        body,
        grid=(nblk,),
        in_specs=[pl.BlockSpec((BR, D_HID), lambda i: (i, 0)),
                  pl.BlockSpec((BR, D_HID), lambda i: (i + nblk, 0)),
                  pl.BlockSpec((BR, 4), lambda i: (i, 0)),
                  pl.BlockSpec((D_HID, D_OUTP), lambda i: (0, 0)),
                  pl.BlockSpec((1, D_OUTP), lambda i: (0, 0))],
        out_specs=pl.BlockSpec((BR, D_OUTP), lambda i: (i, 0)),
        out_shape=jax.ShapeDtypeStruct((N_PAD, D_OUTP), jnp.float32),
    )(parts, parts, degs_t, W2p, b2d)


def kernel(features, edge_index, W0, b0, W1, b1, W2, b2):
    src = edge_index[0]
    dst = edge_index[1]
    pad = N_NODES + jnp.arange(E_PAD - E, dtype=jnp.int32) % (N_PAD - N_NODES)
    src2 = jnp.concatenate([src, pad]).reshape(NW * C, K)
    dst2 = jnp.concatenate([dst, pad]).reshape(NW * C, K)

    feats_p = jnp.zeros((N_PAD, D_HID), jnp.float32).at[:N_NODES].set(features)
    zeros1 = jnp.zeros((RPT,), jnp.float32)
    zrows_h = jnp.zeros((RPT, D_HID), jnp.float32)
    W2p = jnp.zeros((D_HID, D_OUTP), jnp.float32).at[:, :D_OUT].set(W2)
    b0d = jnp.reshape(b0, (1, D_HID))
    b1d = jnp.reshape(b1, (1, D_HID))
    b2d = jnp.zeros((1, D_OUTP), jnp.float32).at[0, :D_OUT].set(b2)

    degs = _degrees(src2, dst2, zeros1)
    degs_t = degs.reshape(4, N_PAD).T

    xs0 = _tc_prelude(feats_p, degs_t)
    parts = _propagate(xs0, src2, dst2, zrows_h, D_HID)
    xs1 = _tc_layer(parts, degs_t, W0, b0d)
    parts = _propagate(xs1, src2, dst2, zrows_h, D_HID)
    xs2 = _tc_layer(parts, degs_t, W1, b1d)
    parts = _propagate(xs2, src2, dst2, zrows_h, D_HID)
    prior_p, xs3 = _tc_layer_prior(parts, degs_t, W1, b1d)
    parts = _propagate(xs3, src2, dst2, zrows_h, D_HID)
    out_p = _tc_final(parts, degs_t, W2p, b2d)

    return out_p[:N_NODES, :D_OUT], prior_p[:N_NODES]


# speedup vs baseline: 1.2466x; 1.2466x over previous
import functools

import jax
import jax.numpy as jnp
from jax import lax
from jax.experimental import pallas as pl
from jax.experimental.pallas import tpu as pltpu
from jax.experimental.pallas import tpu_sc as plsc

N_NODES = 10000
D_HID = 128
D_OUT = 40
D_OUTP = 48
NC, NS = 2, 16
NW = NC * NS
K = 128
N_PAD = 10240
RPT = N_PAD // NS
E = 320000
C = 80
E_PAD = NW * C * K
BR = 512
NBUF = 2
BC = 16
NBLK = C // BC


def _sc_mesh():
    return plsc.VectorSubcoreMesh(core_axis_name="c", subcore_axis_name="s")


def _degrees(src2, dst2, zeros1):
    @functools.partial(
        pl.kernel,
        out_type=jax.ShapeDtypeStruct((4 * N_PAD,), jnp.float32),
        mesh=_sc_mesh(),
        scratch_types=[
            pltpu.VMEM((C, K), jnp.int32),
            pltpu.VMEM((C, K), jnp.int32),
            pltpu.VMEM((K,), jnp.float32),
            pltpu.VMEM_SHARED((N_PAD,), jnp.float32),
            pltpu.VMEM_SHARED((N_PAD,), jnp.float32),
        ],
    )
    def k(src_hbm, dst_hbm, z_hbm, out_hbm, idx_s, idx_d, ones_v, acc_o, acc_i):
        c = lax.axis_index("c")
        s = lax.axis_index("s")
        t = c * NS + s
        pltpu.sync_copy(src_hbm.at[pl.ds(t * C, C)], idx_s)
        pltpu.sync_copy(dst_hbm.at[pl.ds(t * C, C)], idx_d)
        for i in range(K // 16):
            ones_v[pl.ds(i * 16, 16)] = jnp.ones((16,), jnp.float32)
        pltpu.sync_copy(z_hbm, acc_o.at[pl.ds(s * RPT, RPT)])
        pltpu.sync_copy(z_hbm, acc_i.at[pl.ds(s * RPT, RPT)])
        plsc.subcore_barrier()

        def body(j, carry):
            pltpu.sync_copy(ones_v, acc_o.at[idx_s.at[j]], add=True)
            pltpu.sync_copy(ones_v, acc_i.at[idx_d.at[j]], add=True)
            return carry

        lax.fori_loop(0, C, body, 0)
        plsc.subcore_barrier()
        pltpu.sync_copy(acc_o.at[pl.ds(s * RPT, RPT)],
                        out_hbm.at[pl.ds(c * N_PAD + s * RPT, RPT)])
        pltpu.sync_copy(acc_i.at[pl.ds(s * RPT, RPT)],
                        out_hbm.at[pl.ds((2 + c) * N_PAD + s * RPT, RPT)])

    return k(src2, dst2, zeros1)


def _propagate(xs, src2, dst2, zrows, d):
    @functools.partial(
        pl.kernel,
        out_type=jax.ShapeDtypeStruct((2 * N_PAD, d), jnp.float32),
        mesh=_sc_mesh(),
        scratch_types=[
            pltpu.VMEM((2, BC, K), jnp.int32),
            pltpu.VMEM((2, BC, K), jnp.int32),
            pltpu.VMEM((NBUF, K, d), jnp.float32),
            pltpu.SemaphoreType.DMA((NBUF,)),
            pltpu.SemaphoreType.DMA((2,)),
            pltpu.VMEM_SHARED((N_PAD, d), jnp.float32),
        ],
    )
    def k(xs_hbm, src_hbm, dst_hbm, z_hbm, out_hbm, sidx, didx, buf, gsem,
          isem, acc):
        c = lax.axis_index("c")
        s = lax.axis_index("s")
        base = (c * NS + s) * C

        def load_idx_block(blk, slot):
            off = base + blk * BC
            pltpu.async_copy(src_hbm.at[pl.ds(off, BC)], sidx.at[slot],
                             isem.at[slot])
            pltpu.async_copy(dst_hbm.at[pl.ds(off, BC)], didx.at[slot],
                             isem.at[slot])

        def wait_idx_block(slot):
            pltpu.make_async_copy(src_hbm.at[pl.ds(base, BC)], sidx.at[slot],
                                  isem.at[slot]).wait()
            pltpu.make_async_copy(src_hbm.at[pl.ds(base, BC)], didx.at[slot],
                                  isem.at[slot]).wait()

        load_idx_block(0, 0)
        load_idx_block(1, 1)
        pltpu.sync_copy(z_hbm, acc.at[pl.ds(s * RPT, RPT)])
        wait_idx_block(0)
        pltpu.async_copy(xs_hbm.at[sidx.at[0, 0]], buf.at[0], gsem.at[0])
        plsc.subcore_barrier()

        def body(j, carry):
            blk = lax.div(j, BC)

            @pl.when(jnp.logical_and(lax.rem(j, BC) == 0,
                                     jnp.logical_and(blk >= 1,
                                                     blk + 1 < NBLK)))
            def _():
                load_idx_block(blk + 1, lax.rem(blk + 1, 2))

            jn = j + 1
            slot_n = lax.rem(lax.div(jn, BC), 2)

            @pl.when(jn < C)
            def _():
                @pl.when(lax.rem(jn, BC) == 0)
                def _():
                    wait_idx_block(slot_n)

                bn = lax.rem(jn, NBUF)
                pltpu.async_copy(xs_hbm.at[sidx.at[slot_n, lax.rem(jn, BC)]],
                                 buf.at[bn], gsem.at[bn])

            b = lax.rem(j, NBUF)
            slot = lax.rem(blk, 2)
            pltpu.make_async_copy(xs_hbm.at[pl.ds(0, K)], buf.at[b],
                                  gsem.at[b]).wait()
            pltpu.sync_copy(buf.at[b], acc.at[didx.at[slot, lax.rem(j, BC)]],
                            add=True)
            return carry

        lax.fori_loop(0, C, body, 0)
        plsc.subcore_barrier()
        pltpu.sync_copy(acc.at[pl.ds(s * RPT, RPT)],
                        out_hbm.at[pl.ds(c * N_PAD + s * RPT, RPT)])

    return k(xs, src2, dst2, zrows)


def _norms(dblk):
    ns = lax.rsqrt(jnp.maximum(dblk[:, 0] + dblk[:, 1], 1.0))
    nd = lax.rsqrt(jnp.maximum(dblk[:, 2] + dblk[:, 3], 1.0))
    return ns, nd


def _tc_prelude(feats_p, degs_t):
    def body(f_ref, d_ref, o_ref):
        ns, _ = _norms(d_ref[...])
        o_ref[...] = f_ref[...] * ns[:, None]

    return pl.pallas_call(
        body,
        grid=(N_PAD // BR,),
        in_specs=[pl.BlockSpec((BR, D_HID), lambda i: (i, 0)),
                  pl.BlockSpec((BR, 4), lambda i: (i, 0))],
        out_specs=pl.BlockSpec((BR, D_HID), lambda i: (i, 0)),
        out_shape=jax.ShapeDtypeStruct((N_PAD, D_HID), jnp.float32),
    )(feats_p, degs_t)


def _tc_layer(parts, degs_t, W, b2d):
    def body(a0, a1, d_ref, w_ref, b_ref, o_ref):
        ns, nd = _norms(d_ref[...])
        agg = (a0[...] + a1[...]) * nd[:, None]
        h = jnp.dot(agg, w_ref[...], preferred_element_type=jnp.float32)
        h = jnp.maximum(h + b_ref[...], 0.0)
        o_ref[...] = h * ns[:, None]

    nblk = N_PAD // BR
    return pl.pallas_call(
        body,
        grid=(nblk,),
        in_specs=[pl.BlockSpec((BR, D_HID), lambda i: (i, 0)),
                  pl.BlockSpec((BR, D_HID), lambda i: (i + nblk, 0)),
                  pl.BlockSpec((BR, 4), lambda i: (i, 0)),
                  pl.BlockSpec((D_HID, D_HID), lambda i: (0, 0)),
                  pl.BlockSpec((1, D_HID), lambda i: (0, 0))],
        out_specs=pl.BlockSpec((BR, D_HID), lambda i: (i, 0)),
        out_shape=jax.ShapeDtypeStruct((N_PAD, D_HID), jnp.float32),
    )(parts, parts, degs_t, W, b2d)


def _tc_layer_prior(parts, degs_t, W, b2d):
    def body(a0, a1, d_ref, w_ref, b_ref, prior_ref, xs_ref):
        ns, nd = _norms(d_ref[...])
        agg = (a0[...] + a1[...]) * nd[:, None]
        h = jnp.dot(agg, w_ref[...], preferred_element_type=jnp.float32)
        h = jnp.maximum(h + b_ref[...], 0.0)
        prior_ref[...] = h
        xs_ref[...] = h * ns[:, None]

    nblk = N_PAD // BR
    return pl.pallas_call(
        body,
        grid=(nblk,),
        in_specs=[pl.BlockSpec((BR, D_HID), lambda i: (i, 0)),
                  pl.BlockSpec((BR, D_HID), lambda i: (i + nblk, 0)),
                  pl.BlockSpec((BR, 4), lambda i: (i, 0)),
                  pl.BlockSpec((D_HID, D_HID), lambda i: (0, 0)),
                  pl.BlockSpec((1, D_HID), lambda i: (0, 0))],
        out_specs=[pl.BlockSpec((BR, D_HID), lambda i: (i, 0)),
                   pl.BlockSpec((BR, D_HID), lambda i: (i, 0))],
        out_shape=[jax.ShapeDtypeStruct((N_PAD, D_HID), jnp.float32),
                   jax.ShapeDtypeStruct((N_PAD, D_HID), jnp.float32)],
    )(parts, parts, degs_t, W, b2d)


def _tc_final(parts, degs_t, W2p, b2d):
    def body(a0, a1, d_ref, w2_ref, b_ref, o_ref):
        _, nd = _norms(d_ref[...])
        agg = (a0[...] + a1[...]) * nd[:, None]
        h = jnp.dot(agg, w2_ref[...], preferred_element_type=jnp.float32)
        o_ref[...] = h + b_ref[...]

    nblk = N_PAD // BR
    return pl.pallas_call(
        body,
        grid=(nblk,),
        in_specs=[pl.BlockSpec((BR, D_HID), lambda i: (i, 0)),
                  pl.BlockSpec((BR, D_HID), lambda i: (i + nblk, 0)),
                  pl.BlockSpec((BR, 4), lambda i: (i, 0)),
                  pl.BlockSpec((D_HID, D_OUTP), lambda i: (0, 0)),
                  pl.BlockSpec((1, D_OUTP), lambda i: (0, 0))],
        out_specs=pl.BlockSpec((BR, D_OUTP), lambda i: (i, 0)),
        out_shape=jax.ShapeDtypeStruct((N_PAD, D_OUTP), jnp.float32),
    )(parts, parts, degs_t, W2p, b2d)


def kernel(features, edge_index, W0, b0, W1, b1, W2, b2):
    src = edge_index[0]
    dst = edge_index[1]
    pad = N_NODES + jnp.arange(E_PAD - E, dtype=jnp.int32) % (N_PAD - N_NODES)
    src2 = jnp.concatenate([src, pad]).reshape(NW * C, K)
    dst2 = jnp.concatenate([dst, pad]).reshape(NW * C, K)

    feats_p = jnp.zeros((N_PAD, D_HID), jnp.float32).at[:N_NODES].set(features)
    zeros1 = jnp.zeros((RPT,), jnp.float32)
    zrows_h = jnp.zeros((RPT, D_HID), jnp.float32)
    W2p = jnp.zeros((D_HID, D_OUTP), jnp.float32).at[:, :D_OUT].set(W2)
    b0d = jnp.reshape(b0, (1, D_HID))
    b1d = jnp.reshape(b1, (1, D_HID))
    b2d = jnp.zeros((1, D_OUTP), jnp.float32).at[0, :D_OUT].set(b2)

    degs = _degrees(src2, dst2, zeros1)
    degs_t = degs.reshape(4, N_PAD).T

    xs0 = _tc_prelude(feats_p, degs_t)
    parts = _propagate(xs0, src2, dst2, zrows_h, D_HID)
    xs1 = _tc_layer(parts, degs_t, W0, b0d)
    parts = _propagate(xs1, src2, dst2, zrows_h, D_HID)
    xs2 = _tc_layer(parts, degs_t, W1, b1d)
    parts = _propagate(xs2, src2, dst2, zrows_h, D_HID)
    prior_p, xs3 = _tc_layer_prior(parts, degs_t, W1, b1d)
    parts = _propagate(xs3, src2, dst2, zrows_h, D_HID)
    out_p = _tc_final(parts, degs_t, W2p, b2d)

    return out_p[:N_NODES, :D_OUT], prior_p[:N_NODES]


# speedup vs baseline: 1.2994x; 1.0423x over previous
import functools

import jax
import jax.numpy as jnp
from jax import lax
from jax.experimental import pallas as pl
from jax.experimental.pallas import tpu as pltpu
from jax.experimental.pallas import tpu_sc as plsc

N_NODES = 10000
D_HID = 128
D_OUT = 40
D_OUTP = 48
NC, NS = 2, 16
NW = NC * NS
K = 128
N_PAD = 10240
RPT = N_PAD // NS
E = 320000
C = 80
E_PAD = NW * C * K
BR = 1024
NBUF = 2
BC = 16
NBLK = C // BC


def _sc_mesh():
    return plsc.VectorSubcoreMesh(core_axis_name="c", subcore_axis_name="s")


def _degrees(src2, dst2, zeros1):
    @functools.partial(
        pl.kernel,
        out_type=jax.ShapeDtypeStruct((4 * N_PAD,), jnp.float32),
        mesh=_sc_mesh(),
        scratch_types=[
            pltpu.VMEM((C, K), jnp.int32),
            pltpu.VMEM((C, K), jnp.int32),
            pltpu.VMEM((K,), jnp.float32),
            pltpu.VMEM_SHARED((N_PAD,), jnp.float32),
            pltpu.VMEM_SHARED((N_PAD,), jnp.float32),
        ],
    )
    def k(src_hbm, dst_hbm, z_hbm, out_hbm, idx_s, idx_d, ones_v, acc_o, acc_i):
        c = lax.axis_index("c")
        s = lax.axis_index("s")
        t = c * NS + s
        pltpu.sync_copy(src_hbm.at[pl.ds(t * C, C)], idx_s)
        pltpu.sync_copy(dst_hbm.at[pl.ds(t * C, C)], idx_d)
        for i in range(K // 16):
            ones_v[pl.ds(i * 16, 16)] = jnp.ones((16,), jnp.float32)
        pltpu.sync_copy(z_hbm, acc_o.at[pl.ds(s * RPT, RPT)])
        pltpu.sync_copy(z_hbm, acc_i.at[pl.ds(s * RPT, RPT)])
        plsc.subcore_barrier()

        def body(j, carry):
            pltpu.sync_copy(ones_v, acc_o.at[idx_s.at[j]], add=True)
            pltpu.sync_copy(ones_v, acc_i.at[idx_d.at[j]], add=True)
            return carry

        lax.fori_loop(0, C, body, 0)
        plsc.subcore_barrier()
        pltpu.sync_copy(acc_o.at[pl.ds(s * RPT, RPT)],
                        out_hbm.at[pl.ds(c * N_PAD + s * RPT, RPT)])
        pltpu.sync_copy(acc_i.at[pl.ds(s * RPT, RPT)],
                        out_hbm.at[pl.ds((2 + c) * N_PAD + s * RPT, RPT)])

    return k(src2, dst2, zeros1)


def _propagate(xs, src2, dst2, zrows, d):
    @functools.partial(
        pl.kernel,
        out_type=jax.ShapeDtypeStruct((2 * N_PAD, d), jnp.float32),
        mesh=_sc_mesh(),
        scratch_types=[
            pltpu.VMEM((2, BC, K), jnp.int32),
            pltpu.VMEM((2, BC, K), jnp.int32),
            pltpu.VMEM((NBUF, K, d), jnp.float32),
            pltpu.SemaphoreType.DMA((NBUF,)),
            pltpu.SemaphoreType.DMA((2,)),
            pltpu.VMEM_SHARED((N_PAD, d), jnp.float32),
        ],
    )
    def k(xs_hbm, src_hbm, dst_hbm, z_hbm, out_hbm, sidx, didx, buf, gsem,
          isem, acc):
        c = lax.axis_index("c")
        s = lax.axis_index("s")
        base = (c * NS + s) * C

        def load_idx_block(blk, slot):
            off = base + blk * BC
            pltpu.async_copy(src_hbm.at[pl.ds(off, BC)], sidx.at[slot],
                             isem.at[slot])
            pltpu.async_copy(dst_hbm.at[pl.ds(off, BC)], didx.at[slot],
                             isem.at[slot])

        def wait_idx_block(slot):
            pltpu.make_async_copy(src_hbm.at[pl.ds(base, BC)], sidx.at[slot],
                                  isem.at[slot]).wait()
            pltpu.make_async_copy(src_hbm.at[pl.ds(base, BC)], didx.at[slot],
                                  isem.at[slot]).wait()

        load_idx_block(0, 0)
        load_idx_block(1, 1)
        pltpu.sync_copy(z_hbm, acc.at[pl.ds(s * RPT, RPT)])
        wait_idx_block(0)
        pltpu.async_copy(xs_hbm.at[sidx.at[0, 0]], buf.at[0], gsem.at[0])
        plsc.subcore_barrier()

        def body(j, carry):
            blk = lax.div(j, BC)

            @pl.when(jnp.logical_and(lax.rem(j, BC) == 0,
                                     jnp.logical_and(blk >= 1,
                                                     blk + 1 < NBLK)))
            def _():
                load_idx_block(blk + 1, lax.rem(blk + 1, 2))

            jn = j + 1
            slot_n = lax.rem(lax.div(jn, BC), 2)

            @pl.when(jn < C)
            def _():
                @pl.when(lax.rem(jn, BC) == 0)
                def _():
                    wait_idx_block(slot_n)

                bn = lax.rem(jn, NBUF)
                pltpu.async_copy(xs_hbm.at[sidx.at[slot_n, lax.rem(jn, BC)]],
                                 buf.at[bn], gsem.at[bn])

            b = lax.rem(j, NBUF)
            slot = lax.rem(blk, 2)
            pltpu.make_async_copy(xs_hbm.at[pl.ds(0, K)], buf.at[b],
                                  gsem.at[b]).wait()
            pltpu.sync_copy(buf.at[b], acc.at[didx.at[slot, lax.rem(j, BC)]],
                            add=True)
            return carry

        lax.fori_loop(0, C, body, 0)
        plsc.subcore_barrier()
        pltpu.sync_copy(acc.at[pl.ds(s * RPT, RPT)],
                        out_hbm.at[pl.ds(c * N_PAD + s * RPT, RPT)])

    return k(xs, src2, dst2, zrows)


def _norms(dblk):
    ns = lax.rsqrt(jnp.maximum(dblk[:, 0] + dblk[:, 1], 1.0))
    nd = lax.rsqrt(jnp.maximum(dblk[:, 2] + dblk[:, 3], 1.0))
    return ns, nd


def _tc_prelude(feats_p, degs_t):
    def body(f_ref, d_ref, o_ref):
        ns, _ = _norms(d_ref[...])
        o_ref[...] = f_ref[...] * ns[:, None]

    return pl.pallas_call(
        body,
        grid=(N_PAD // BR,),
        in_specs=[pl.BlockSpec((BR, D_HID), lambda i: (i, 0)),
                  pl.BlockSpec((BR, 4), lambda i: (i, 0))],
        out_specs=pl.BlockSpec((BR, D_HID), lambda i: (i, 0)),
        out_shape=jax.ShapeDtypeStruct((N_PAD, D_HID), jnp.float32),
    )(feats_p, degs_t)


def _tc_layer(parts, degs_t, W, b2d):
    def body(a0, a1, d_ref, w_ref, b_ref, o_ref):
        ns, nd = _norms(d_ref[...])
        agg = (a0[...] + a1[...]) * nd[:, None]
        h = jnp.dot(agg, w_ref[...], preferred_element_type=jnp.float32)
        h = jnp.maximum(h + b_ref[...], 0.0)
        o_ref[...] = h * ns[:, None]

    nblk = N_PAD // BR
    return pl.pallas_call(
        body,
        grid=(nblk,),
        in_specs=[pl.BlockSpec((BR, D_HID), lambda i: (i, 0)),
                  pl.BlockSpec((BR, D_HID), lambda i: (i + nblk, 0)),
                  pl.BlockSpec((BR, 4), lambda i: (i, 0)),
                  pl.BlockSpec((D_HID, D_HID), lambda i: (0, 0)),
                  pl.BlockSpec((1, D_HID), lambda i: (0, 0))],
        out_specs=pl.BlockSpec((BR, D_HID), lambda i: (i, 0)),
        out_shape=jax.ShapeDtypeStruct((N_PAD, D_HID), jnp.float32),
    )(parts, parts, degs_t, W, b2d)


def _tc_layer_prior(parts, degs_t, W, b2d):
    def body(a0, a1, d_ref, w_ref, b_ref, prior_ref, xs_ref):
        ns, nd = _norms(d_ref[...])
        agg = (a0[...] + a1[...]) * nd[:, None]
        h = jnp.dot(agg, w_ref[...], preferred_element_type=jnp.float32)
        h = jnp.maximum(h + b_ref[...], 0.0)
        prior_ref[...] = h
        xs_ref[...] = h * ns[:, None]

    nblk = N_PAD // BR
    return pl.pallas_call(
        body,
        grid=(nblk,),
        in_specs=[pl.BlockSpec((BR, D_HID), lambda i: (i, 0)),
                  pl.BlockSpec((BR, D_HID), lambda i: (i + nblk, 0)),
                  pl.BlockSpec((BR, 4), lambda i: (i, 0)),
                  pl.BlockSpec((D_HID, D_HID), lambda i: (0, 0)),
                  pl.BlockSpec((1, D_HID), lambda i: (0, 0))],
        out_specs=[pl.BlockSpec((BR, D_HID), lambda i: (i, 0)),
                   pl.BlockSpec((BR, D_HID), lambda i: (i, 0))],
        out_shape=[jax.ShapeDtypeStruct((N_PAD, D_HID), jnp.float32),
                   jax.ShapeDtypeStruct((N_PAD, D_HID), jnp.float32)],
    )(parts, parts, degs_t, W, b2d)


def _tc_final(parts, degs_t, W2p, b2d):
    def body(a0, a1, d_ref, w2_ref, b_ref, o_ref):
        _, nd = _norms(d_ref[...])
        agg = (a0[...] + a1[...]) * nd[:, None]
        h = jnp.dot(agg, w2_ref[...], preferred_element_type=jnp.float32)
        o_ref[...] = h + b_ref[...]

    nblk = N_PAD // BR
    return pl.pallas_call(
        body,
        grid=(nblk,),
        in_specs=[pl.BlockSpec((BR, D_HID), lambda i: (i, 0)),
                  pl.BlockSpec((BR, D_HID), lambda i: (i + nblk, 0)),
                  pl.BlockSpec((BR, 4), lambda i: (i, 0)),
                  pl.BlockSpec((D_HID, D_OUTP), lambda i: (0, 0)),
                  pl.BlockSpec((1, D_OUTP), lambda i: (0, 0))],
        out_specs=pl.BlockSpec((BR, D_OUTP), lambda i: (i, 0)),
        out_shape=jax.ShapeDtypeStruct((N_PAD, D_OUTP), jnp.float32),
    )(parts, parts, degs_t, W2p, b2d)


def kernel(features, edge_index, W0, b0, W1, b1, W2, b2):
    src = edge_index[0]
    dst = edge_index[1]
    pad = N_NODES + jnp.arange(E_PAD - E, dtype=jnp.int32) % (N_PAD - N_NODES)
    src2 = jnp.concatenate([src, pad]).reshape(NW * C, K)
    dst2 = jnp.concatenate([dst, pad]).reshape(NW * C, K)

    feats_p = jnp.zeros((N_PAD, D_HID), jnp.float32).at[:N_NODES].set(features)
    zeros1 = jnp.zeros((RPT,), jnp.float32)
    zrows_h = jnp.zeros((RPT, D_HID), jnp.float32)
    W2p = jnp.zeros((D_HID, D_OUTP), jnp.float32).at[:, :D_OUT].set(W2)
    b0d = jnp.reshape(b0, (1, D_HID))
    b1d = jnp.reshape(b1, (1, D_HID))
    b2d = jnp.zeros((1, D_OUTP), jnp.float32).at[0, :D_OUT].set(b2)

    degs = _degrees(src2, dst2, zeros1)
    degs_t = degs.reshape(4, N_PAD).T

    xs0 = _tc_prelude(feats_p, degs_t)
    parts = _propagate(xs0, src2, dst2, zrows_h, D_HID)
    xs1 = _tc_layer(parts, degs_t, W0, b0d)
    parts = _propagate(xs1, src2, dst2, zrows_h, D_HID)
    xs2 = _tc_layer(parts, degs_t, W1, b1d)
    parts = _propagate(xs2, src2, dst2, zrows_h, D_HID)
    prior_p, xs3 = _tc_layer_prior(parts, degs_t, W1, b1d)
    parts = _propagate(xs3, src2, dst2, zrows_h, D_HID)
    out_p = _tc_final(parts, degs_t, W2p, b2d)

    return out_p[:N_NODES, :D_OUT], prior_p[:N_NODES]


# speedup vs baseline: 1.3269x; 1.0212x over previous
import functools

import jax
import jax.numpy as jnp
from jax import lax
from jax.experimental import pallas as pl
from jax.experimental.pallas import tpu as pltpu
from jax.experimental.pallas import tpu_sc as plsc

N_NODES = 10000
D_HID = 128
D_OUT = 40
D_OUTP = 48
NC, NS = 2, 16
NW = NC * NS
K = 128
N_PAD = 10240
RPT = N_PAD // NS
E = 320000
C = 80
E_PAD = NW * C * K
BR = 2048
NBUF = 2
BC = 16
NBLK = C // BC


def _sc_mesh():
    return plsc.VectorSubcoreMesh(core_axis_name="c", subcore_axis_name="s")


def _degrees(src2, dst2, zeros1):
    @functools.partial(
        pl.kernel,
        out_type=jax.ShapeDtypeStruct((4 * N_PAD,), jnp.float32),
        mesh=_sc_mesh(),
        scratch_types=[
            pltpu.VMEM((C, K), jnp.int32),
            pltpu.VMEM((C, K), jnp.int32),
            pltpu.VMEM((K,), jnp.float32),
            pltpu.VMEM_SHARED((N_PAD,), jnp.float32),
            pltpu.VMEM_SHARED((N_PAD,), jnp.float32),
        ],
    )
    def k(src_hbm, dst_hbm, z_hbm, out_hbm, idx_s, idx_d, ones_v, acc_o, acc_i):
        c = lax.axis_index("c")
        s = lax.axis_index("s")
        t = c * NS + s
        pltpu.sync_copy(src_hbm.at[pl.ds(t * C, C)], idx_s)
        pltpu.sync_copy(dst_hbm.at[pl.ds(t * C, C)], idx_d)
        for i in range(K // 16):
            ones_v[pl.ds(i * 16, 16)] = jnp.ones((16,), jnp.float32)
        pltpu.sync_copy(z_hbm, acc_o.at[pl.ds(s * RPT, RPT)])
        pltpu.sync_copy(z_hbm, acc_i.at[pl.ds(s * RPT, RPT)])
        plsc.subcore_barrier()

        def body(j, carry):
            pltpu.sync_copy(ones_v, acc_o.at[idx_s.at[j]], add=True)
            pltpu.sync_copy(ones_v, acc_i.at[idx_d.at[j]], add=True)
            return carry

        lax.fori_loop(0, C, body, 0)
        plsc.subcore_barrier()
        pltpu.sync_copy(acc_o.at[pl.ds(s * RPT, RPT)],
                        out_hbm.at[pl.ds(c * N_PAD + s * RPT, RPT)])
        pltpu.sync_copy(acc_i.at[pl.ds(s * RPT, RPT)],
                        out_hbm.at[pl.ds((2 + c) * N_PAD + s * RPT, RPT)])

    return k(src2, dst2, zeros1)


def _propagate(xs, src2, dst2, zrows, d):
    @functools.partial(
        pl.kernel,
        out_type=jax.ShapeDtypeStruct((2 * N_PAD, d), jnp.float32),
        mesh=_sc_mesh(),
        scratch_types=[
            pltpu.VMEM((2, BC, K), jnp.int32),
            pltpu.VMEM((2, BC, K), jnp.int32),
            pltpu.VMEM((NBUF, K, d), jnp.float32),
            pltpu.SemaphoreType.DMA((NBUF,)),
            pltpu.SemaphoreType.DMA((2,)),
            pltpu.VMEM_SHARED((N_PAD, d), jnp.float32),
        ],
    )
    def k(xs_hbm, src_hbm, dst_hbm, z_hbm, out_hbm, sidx, didx, buf, gsem,
          isem, acc):
        c = lax.axis_index("c")
        s = lax.axis_index("s")
        base = (c * NS + s) * C

        def load_idx_block(blk, slot):
            off = base + blk * BC
            pltpu.async_copy(src_hbm.at[pl.ds(off, BC)], sidx.at[slot],
                             isem.at[slot])
            pltpu.async_copy(dst_hbm.at[pl.ds(off, BC)], didx.at[slot],
                             isem.at[slot])

        def wait_idx_block(slot):
            pltpu.make_async_copy(src_hbm.at[pl.ds(base, BC)], sidx.at[slot],
                                  isem.at[slot]).wait()
            pltpu.make_async_copy(src_hbm.at[pl.ds(base, BC)], didx.at[slot],
                                  isem.at[slot]).wait()

        load_idx_block(0, 0)
        load_idx_block(1, 1)
        pltpu.sync_copy(z_hbm, acc.at[pl.ds(s * RPT, RPT)])
        wait_idx_block(0)
        pltpu.async_copy(xs_hbm.at[sidx.at[0, 0]], buf.at[0], gsem.at[0])
        plsc.subcore_barrier()

        def body(j, carry):
            blk = lax.div(j, BC)

            @pl.when(jnp.logical_and(lax.rem(j, BC) == 0,
                                     jnp.logical_and(blk >= 1,
                                                     blk + 1 < NBLK)))
            def _():
                load_idx_block(blk + 1, lax.rem(blk + 1, 2))

            jn = j + 1
            slot_n = lax.rem(lax.div(jn, BC), 2)

            @pl.when(jn < C)
            def _():
                @pl.when(lax.rem(jn, BC) == 0)
                def _():
                    wait_idx_block(slot_n)

                bn = lax.rem(jn, NBUF)
                pltpu.async_copy(xs_hbm.at[sidx.at[slot_n, lax.rem(jn, BC)]],
                                 buf.at[bn], gsem.at[bn])

            b = lax.rem(j, NBUF)
            slot = lax.rem(blk, 2)
            pltpu.make_async_copy(xs_hbm.at[pl.ds(0, K)], buf.at[b],
                                  gsem.at[b]).wait()
            pltpu.sync_copy(buf.at[b], acc.at[didx.at[slot, lax.rem(j, BC)]],
                            add=True)
            return carry

        lax.fori_loop(0, C, body, 0)
        plsc.subcore_barrier()
        pltpu.sync_copy(acc.at[pl.ds(s * RPT, RPT)],
                        out_hbm.at[pl.ds(c * N_PAD + s * RPT, RPT)])

    return k(xs, src2, dst2, zrows)


def _norms(dblk):
    ns = lax.rsqrt(jnp.maximum(dblk[:, 0] + dblk[:, 1], 1.0))
    nd = lax.rsqrt(jnp.maximum(dblk[:, 2] + dblk[:, 3], 1.0))
    return ns, nd


def _tc_prelude(feats_p, degs_t):
    def body(f_ref, d_ref, o_ref):
        ns, _ = _norms(d_ref[...])
        o_ref[...] = f_ref[...] * ns[:, None]

    return pl.pallas_call(
        body,
        grid=(N_PAD // BR,),
        in_specs=[pl.BlockSpec((BR, D_HID), lambda i: (i, 0)),
                  pl.BlockSpec((BR, 4), lambda i: (i, 0))],
        out_specs=pl.BlockSpec((BR, D_HID), lambda i: (i, 0)),
        out_shape=jax.ShapeDtypeStruct((N_PAD, D_HID), jnp.float32),
    )(feats_p, degs_t)


def _tc_layer(parts, degs_t, W, b2d):
    def body(a0, a1, d_ref, w_ref, b_ref, o_ref):
        ns, nd = _norms(d_ref[...])
        agg = (a0[...] + a1[...]) * nd[:, None]
        h = jnp.dot(agg, w_ref[...], preferred_element_type=jnp.float32)
        h = jnp.maximum(h + b_ref[...], 0.0)
        o_ref[...] = h * ns[:, None]

    nblk = N_PAD // BR
    return pl.pallas_call(
        body,
        grid=(nblk,),
        in_specs=[pl.BlockSpec((BR, D_HID), lambda i: (i, 0)),
                  pl.BlockSpec((BR, D_HID), lambda i: (i + nblk, 0)),
                  pl.BlockSpec((BR, 4), lambda i: (i, 0)),
                  pl.BlockSpec((D_HID, D_HID), lambda i: (0, 0)),
                  pl.BlockSpec((1, D_HID), lambda i: (0, 0))],
        out_specs=pl.BlockSpec((BR, D_HID), lambda i: (i, 0)),
        out_shape=jax.ShapeDtypeStruct((N_PAD, D_HID), jnp.float32),
    )(parts, parts, degs_t, W, b2d)


def _tc_layer_prior(parts, degs_t, W, b2d):
    def body(a0, a1, d_ref, w_ref, b_ref, prior_ref, xs_ref):
        ns, nd = _norms(d_ref[...])
        agg = (a0[...] + a1[...]) * nd[:, None]
        h = jnp.dot(agg, w_ref[...], preferred_element_type=jnp.float32)
        h = jnp.maximum(h + b_ref[...], 0.0)
        prior_ref[...] = h
        xs_ref[...] = h * ns[:, None]

    nblk = N_PAD // BR
    return pl.pallas_call(
        body,
        grid=(nblk,),
        in_specs=[pl.BlockSpec((BR, D_HID), lambda i: (i, 0)),
                  pl.BlockSpec((BR, D_HID), lambda i: (i + nblk, 0)),
                  pl.BlockSpec((BR, 4), lambda i: (i, 0)),
                  pl.BlockSpec((D_HID, D_HID), lambda i: (0, 0)),
                  pl.BlockSpec((1, D_HID), lambda i: (0, 0))],
        out_specs=[pl.BlockSpec((BR, D_HID), lambda i: (i, 0)),
                   pl.BlockSpec((BR, D_HID), lambda i: (i, 0))],
        out_shape=[jax.ShapeDtypeStruct((N_PAD, D_HID), jnp.float32),
                   jax.ShapeDtypeStruct((N_PAD, D_HID), jnp.float32)],
    )(parts, parts, degs_t, W, b2d)


def _tc_final(parts, degs_t, W2p, b2d):
    def body(a0, a1, d_ref, w2_ref, b_ref, o_ref):
        _, nd = _norms(d_ref[...])
        agg = (a0[...] + a1[...]) * nd[:, None]
        h = jnp.dot(agg, w2_ref[...], preferred_element_type=jnp.float32)
        o_ref[...] = h + b_ref[...]

    nblk = N_PAD // BR
    return pl.pallas_call(
        body,
        grid=(nblk,),
        in_specs=[pl.BlockSpec((BR, D_HID), lambda i: (i, 0)),
                  pl.BlockSpec((BR, D_HID), lambda i: (i + nblk, 0)),
                  pl.BlockSpec((BR, 4), lambda i: (i, 0)),
                  pl.BlockSpec((D_HID, D_OUTP), lambda i: (0, 0)),
                  pl.BlockSpec((1, D_OUTP), lambda i: (0, 0))],
        out_specs=pl.BlockSpec((BR, D_OUTP), lambda i: (i, 0)),
        out_shape=jax.ShapeDtypeStruct((N_PAD, D_OUTP), jnp.float32),
    )(parts, parts, degs_t, W2p, b2d)


def kernel(features, edge_index, W0, b0, W1, b1, W2, b2):
    src = edge_index[0]
    dst = edge_index[1]
    pad = N_NODES + jnp.arange(E_PAD - E, dtype=jnp.int32) % (N_PAD - N_NODES)
    src2 = jnp.concatenate([src, pad]).reshape(NW * C, K)
    dst2 = jnp.concatenate([dst, pad]).reshape(NW * C, K)

    feats_p = jnp.zeros((N_PAD, D_HID), jnp.float32).at[:N_NODES].set(features)
    zeros1 = jnp.zeros((RPT,), jnp.float32)
    zrows_h = jnp.zeros((RPT, D_HID), jnp.float32)
    W2p = jnp.zeros((D_HID, D_OUTP), jnp.float32).at[:, :D_OUT].set(W2)
    b0d = jnp.reshape(b0, (1, D_HID))
    b1d = jnp.reshape(b1, (1, D_HID))
    b2d = jnp.zeros((1, D_OUTP), jnp.float32).at[0, :D_OUT].set(b2)

    degs = _degrees(src2, dst2, zeros1)
    degs_t = degs.reshape(4, N_PAD).T

    xs0 = _tc_prelude(feats_p, degs_t)
    parts = _propagate(xs0, src2, dst2, zrows_h, D_HID)
    xs1 = _tc_layer(parts, degs_t, W0, b0d)
    parts = _propagate(xs1, src2, dst2, zrows_h, D_HID)
    xs2 = _tc_layer(parts, degs_t, W1, b1d)
    parts = _propagate(xs2, src2, dst2, zrows_h, D_HID)
    prior_p, xs3 = _tc_layer_prior(parts, degs_t, W1, b1d)
    parts = _propagate(xs3, src2, dst2, zrows_h, D_HID)
    out_p = _tc_final(parts, degs_t, W2p, b2d)

    return out_p[:N_NODES, :D_OUT], prior_p[:N_NODES]


# speedup vs baseline: 1.3498x; 1.0172x over previous
import functools

import jax
import jax.numpy as jnp
from jax import lax
from jax.experimental import pallas as pl
from jax.experimental.pallas import tpu as pltpu
from jax.experimental.pallas import tpu_sc as plsc

N_NODES = 10000
D_HID = 128
D_OUT = 40
D_OUTP = 48
NC, NS = 2, 16
NW = NC * NS
K = 128
N_PAD = 10240
RPT = N_PAD // NS
E = 320000
C = 80
E_PAD = NW * C * K
BR = 5120
NBUF = 2
BC = 16
NBLK = C // BC


def _sc_mesh():
    return plsc.VectorSubcoreMesh(core_axis_name="c", subcore_axis_name="s")


def _degrees(src2, dst2, zeros1):
    @functools.partial(
        pl.kernel,
        out_type=jax.ShapeDtypeStruct((4 * N_PAD,), jnp.float32),
        mesh=_sc_mesh(),
        scratch_types=[
            pltpu.VMEM((C, K), jnp.int32),
            pltpu.VMEM((C, K), jnp.int32),
            pltpu.VMEM((K,), jnp.float32),
            pltpu.VMEM_SHARED((N_PAD,), jnp.float32),
            pltpu.VMEM_SHARED((N_PAD,), jnp.float32),
        ],
    )
    def k(src_hbm, dst_hbm, z_hbm, out_hbm, idx_s, idx_d, ones_v, acc_o, acc_i):
        c = lax.axis_index("c")
        s = lax.axis_index("s")
        t = c * NS + s
        pltpu.sync_copy(src_hbm.at[pl.ds(t * C, C)], idx_s)
        pltpu.sync_copy(dst_hbm.at[pl.ds(t * C, C)], idx_d)
        for i in range(K // 16):
            ones_v[pl.ds(i * 16, 16)] = jnp.ones((16,), jnp.float32)
        pltpu.sync_copy(z_hbm, acc_o.at[pl.ds(s * RPT, RPT)])
        pltpu.sync_copy(z_hbm, acc_i.at[pl.ds(s * RPT, RPT)])
        plsc.subcore_barrier()

        def body(j, carry):
            pltpu.sync_copy(ones_v, acc_o.at[idx_s.at[j]], add=True)
            pltpu.sync_copy(ones_v, acc_i.at[idx_d.at[j]], add=True)
            return carry

        lax.fori_loop(0, C, body, 0)
        plsc.subcore_barrier()
        pltpu.sync_copy(acc_o.at[pl.ds(s * RPT, RPT)],
                        out_hbm.at[pl.ds(c * N_PAD + s * RPT, RPT)])
        pltpu.sync_copy(acc_i.at[pl.ds(s * RPT, RPT)],
                        out_hbm.at[pl.ds((2 + c) * N_PAD + s * RPT, RPT)])

    return k(src2, dst2, zeros1)


def _propagate(xs, src2, dst2, zrows, d):
    @functools.partial(
        pl.kernel,
        out_type=jax.ShapeDtypeStruct((2 * N_PAD, d), jnp.float32),
        mesh=_sc_mesh(),
        scratch_types=[
            pltpu.VMEM((2, BC, K), jnp.int32),
            pltpu.VMEM((2, BC, K), jnp.int32),
            pltpu.VMEM((NBUF, K, d), jnp.float32),
            pltpu.SemaphoreType.DMA((NBUF,)),
            pltpu.SemaphoreType.DMA((2,)),
            pltpu.VMEM_SHARED((N_PAD, d), jnp.float32),
        ],
    )
    def k(xs_hbm, src_hbm, dst_hbm, z_hbm, out_hbm, sidx, didx, buf, gsem,
          isem, acc):
        c = lax.axis_index("c")
        s = lax.axis_index("s")
        base = (c * NS + s) * C

        def load_idx_block(blk, slot):
            off = base + blk * BC
            pltpu.async_copy(src_hbm.at[pl.ds(off, BC)], sidx.at[slot],
                             isem.at[slot])
            pltpu.async_copy(dst_hbm.at[pl.ds(off, BC)], didx.at[slot],
                             isem.at[slot])

        def wait_idx_block(slot):
            pltpu.make_async_copy(src_hbm.at[pl.ds(base, BC)], sidx.at[slot],
                                  isem.at[slot]).wait()
            pltpu.make_async_copy(src_hbm.at[pl.ds(base, BC)], didx.at[slot],
                                  isem.at[slot]).wait()

        load_idx_block(0, 0)
        load_idx_block(1, 1)
        pltpu.sync_copy(z_hbm, acc.at[pl.ds(s * RPT, RPT)])
        wait_idx_block(0)
        pltpu.async_copy(xs_hbm.at[sidx.at[0, 0]], buf.at[0], gsem.at[0])
        plsc.subcore_barrier()

        def body(j, carry):
            blk = lax.div(j, BC)

            @pl.when(jnp.logical_and(lax.rem(j, BC) == 0,
                                     jnp.logical_and(blk >= 1,
                                                     blk + 1 < NBLK)))
            def _():
                load_idx_block(blk + 1, lax.rem(blk + 1, 2))

            jn = j + 1
            slot_n = lax.rem(lax.div(jn, BC), 2)

            @pl.when(jn < C)
            def _():
                @pl.when(lax.rem(jn, BC) == 0)
                def _():
                    wait_idx_block(slot_n)

                bn = lax.rem(jn, NBUF)
                pltpu.async_copy(xs_hbm.at[sidx.at[slot_n, lax.rem(jn, BC)]],
                                 buf.at[bn], gsem.at[bn])

            b = lax.rem(j, NBUF)
            slot = lax.rem(blk, 2)
            pltpu.make_async_copy(xs_hbm.at[pl.ds(0, K)], buf.at[b],
                                  gsem.at[b]).wait()
            pltpu.sync_copy(buf.at[b], acc.at[didx.at[slot, lax.rem(j, BC)]],
                            add=True)
            return carry

        lax.fori_loop(0, C, body, 0)
        plsc.subcore_barrier()
        pltpu.sync_copy(acc.at[pl.ds(s * RPT, RPT)],
                        out_hbm.at[pl.ds(c * N_PAD + s * RPT, RPT)])

    return k(xs, src2, dst2, zrows)


def _norms(dblk):
    ns = lax.rsqrt(jnp.maximum(dblk[:, 0] + dblk[:, 1], 1.0))
    nd = lax.rsqrt(jnp.maximum(dblk[:, 2] + dblk[:, 3], 1.0))
    return ns, nd


def _tc_prelude(feats_p, degs_t):
    def body(f_ref, d_ref, o_ref):
        ns, _ = _norms(d_ref[...])
        o_ref[...] = f_ref[...] * ns[:, None]

    return pl.pallas_call(
        body,
        grid=(N_PAD // BR,),
        in_specs=[pl.BlockSpec((BR, D_HID), lambda i: (i, 0)),
                  pl.BlockSpec((BR, 4), lambda i: (i, 0))],
        out_specs=pl.BlockSpec((BR, D_HID), lambda i: (i, 0)),
        out_shape=jax.ShapeDtypeStruct((N_PAD, D_HID), jnp.float32),
    )(feats_p, degs_t)


def _tc_layer(parts, degs_t, W, b2d):
    def body(a0, a1, d_ref, w_ref, b_ref, o_ref):
        ns, nd = _norms(d_ref[...])
        agg = (a0[...] + a1[...]) * nd[:, None]
        h = jnp.dot(agg, w_ref[...], preferred_element_type=jnp.float32)
        h = jnp.maximum(h + b_ref[...], 0.0)
        o_ref[...] = h * ns[:, None]

    nblk = N_PAD // BR
    return pl.pallas_call(
        body,
        grid=(nblk,),
        in_specs=[pl.BlockSpec((BR, D_HID), lambda i: (i, 0)),
                  pl.BlockSpec((BR, D_HID), lambda i: (i + nblk, 0)),
                  pl.BlockSpec((BR, 4), lambda i: (i, 0)),
                  pl.BlockSpec((D_HID, D_HID), lambda i: (0, 0)),
                  pl.BlockSpec((1, D_HID), lambda i: (0, 0))],
        out_specs=pl.BlockSpec((BR, D_HID), lambda i: (i, 0)),
        out_shape=jax.ShapeDtypeStruct((N_PAD, D_HID), jnp.float32),
    )(parts, parts, degs_t, W, b2d)


def _tc_layer_prior(parts, degs_t, W, b2d):
    def body(a0, a1, d_ref, w_ref, b_ref, prior_ref, xs_ref):
        ns, nd = _norms(d_ref[...])
        agg = (a0[...] + a1[...]) * nd[:, None]
        h = jnp.dot(agg, w_ref[...], preferred_element_type=jnp.float32)
        h = jnp.maximum(h + b_ref[...], 0.0)
        prior_ref[...] = h
        xs_ref[...] = h * ns[:, None]

    nblk = N_PAD // BR
    return pl.pallas_call(
        body,
        grid=(nblk,),
        in_specs=[pl.BlockSpec((BR, D_HID), lambda i: (i, 0)),
                  pl.BlockSpec((BR, D_HID), lambda i: (i + nblk, 0)),
                  pl.BlockSpec((BR, 4), lambda i: (i, 0)),
                  pl.BlockSpec((D_HID, D_HID), lambda i: (0, 0)),
                  pl.BlockSpec((1, D_HID), lambda i: (0, 0))],
        out_specs=[pl.BlockSpec((BR, D_HID), lambda i: (i, 0)),
                   pl.BlockSpec((BR, D_HID), lambda i: (i, 0))],
        out_shape=[jax.ShapeDtypeStruct((N_PAD, D_HID), jnp.float32),
                   jax.ShapeDtypeStruct((N_PAD, D_HID), jnp.float32)],
    )(parts, parts, degs_t, W, b2d)


def _tc_final(parts, degs_t, W2p, b2d):
    def body(a0, a1, d_ref, w2_ref, b_ref, o_ref):
        _, nd = _norms(d_ref[...])
        agg = (a0[...] + a1[...]) * nd[:, None]
        h = jnp.dot(agg, w2_ref[...], preferred_element_type=jnp.float32)
        o_ref[...] = h + b_ref[...]

    nblk = N_PAD // BR
    return pl.pallas_call(
        body,
        grid=(nblk,),
        in_specs=[pl.BlockSpec((BR, D_HID), lambda i: (i, 0)),
                  pl.BlockSpec((BR, D_HID), lambda i: (i + nblk, 0)),
                  pl.BlockSpec((BR, 4), lambda i: (i, 0)),
                  pl.BlockSpec((D_HID, D_OUTP), lambda i: (0, 0)),
                  pl.BlockSpec((1, D_OUTP), lambda i: (0, 0))],
        out_specs=pl.BlockSpec((BR, D_OUTP), lambda i: (i, 0)),
        out_shape=jax.ShapeDtypeStruct((N_PAD, D_OUTP), jnp.float32),
    )(parts, parts, degs_t, W2p, b2d)


def kernel(features, edge_index, W0, b0, W1, b1, W2, b2):
    src = edge_index[0]
    dst = edge_index[1]
    pad = N_NODES + jnp.arange(E_PAD - E, dtype=jnp.int32) % (N_PAD - N_NODES)
    src2 = jnp.concatenate([src, pad]).reshape(NW * C, K)
    dst2 = jnp.concatenate([dst, pad]).reshape(NW * C, K)

    feats_p = jnp.zeros((N_PAD, D_HID), jnp.float32).at[:N_NODES].set(features)
    zeros1 = jnp.zeros((RPT,), jnp.float32)
    zrows_h = jnp.zeros((RPT, D_HID), jnp.float32)
    W2p = jnp.zeros((D_HID, D_OUTP), jnp.float32).at[:, :D_OUT].set(W2)
    b0d = jnp.reshape(b0, (1, D_HID))
    b1d = jnp.reshape(b1, (1, D_HID))
    b2d = jnp.zeros((1, D_OUTP), jnp.float32).at[0, :D_OUT].set(b2)

    degs = _degrees(src2, dst2, zeros1)
    degs_t = degs.reshape(4, N_PAD).T

    xs0 = _tc_prelude(feats_p, degs_t)
    parts = _propagate(xs0, src2, dst2, zrows_h, D_HID)
    xs1 = _tc_layer(parts, degs_t, W0, b0d)
    parts = _propagate(xs1, src2, dst2, zrows_h, D_HID)
    xs2 = _tc_layer(parts, degs_t, W1, b1d)
    parts = _propagate(xs2, src2, dst2, zrows_h, D_HID)
    prior_p, xs3 = _tc_layer_prior(parts, degs_t, W1, b1d)
    parts = _propagate(xs3, src2, dst2, zrows_h, D_HID)
    out_p = _tc_final(parts, degs_t, W2p, b2d)

    return out_p[:N_NODES, :D_OUT], prior_p[:N_NODES]


# speedup vs baseline: 1.3776x; 1.0206x over previous
import functools

import jax
import jax.numpy as jnp
from jax import lax
from jax.experimental import pallas as pl
from jax.experimental.pallas import tpu as pltpu
from jax.experimental.pallas import tpu_sc as plsc

N_NODES = 10000
D_HID = 128
D_OUT = 40
D_OUTP = 48
NC, NS = 2, 16
NW = NC * NS
K = 128
N_PAD = 10240
RPT = N_PAD // NS
E = 320000
C = 80
E_PAD = NW * C * K
BR = 5120
NBUF = 2
BC = 16
NBLK = C // BC


def _sc_mesh():
    return plsc.VectorSubcoreMesh(core_axis_name="c", subcore_axis_name="s")


def _degrees(src2, dst2, zeros1):
    @functools.partial(
        pl.kernel,
        out_type=jax.ShapeDtypeStruct((4 * N_PAD,), jnp.float32),
        mesh=_sc_mesh(),
        scratch_types=[
            pltpu.VMEM((C, K), jnp.int32),
            pltpu.VMEM((C, K), jnp.int32),
            pltpu.VMEM((K,), jnp.float32),
            pltpu.SemaphoreType.DMA((4,)),
            pltpu.VMEM_SHARED((N_PAD,), jnp.float32),
            pltpu.VMEM_SHARED((N_PAD,), jnp.float32),
        ],
    )
    def k(src_hbm, dst_hbm, z_hbm, out_hbm, idx_s, idx_d, ones_v, dsem,
          acc_o, acc_i):
        c = lax.axis_index("c")
        s = lax.axis_index("s")
        t = c * NS + s
        pltpu.sync_copy(src_hbm.at[pl.ds(t * C, C)], idx_s)
        pltpu.sync_copy(dst_hbm.at[pl.ds(t * C, C)], idx_d)
        for i in range(K // 16):
            ones_v[pl.ds(i * 16, 16)] = jnp.ones((16,), jnp.float32)
        pltpu.sync_copy(z_hbm, acc_o.at[pl.ds(s * RPT, RPT)])
        pltpu.sync_copy(z_hbm, acc_i.at[pl.ds(s * RPT, RPT)])
        plsc.subcore_barrier()

        def issue(j):
            b = lax.rem(j, 4)
            pltpu.async_copy(ones_v, acc_o.at[idx_s.at[j]], dsem.at[b],
                             add=True)
            pltpu.async_copy(ones_v, acc_i.at[idx_d.at[j]], dsem.at[b],
                             add=True)

        for j in range(3):
            issue(j)

        def body(j, carry):
            @pl.when(j + 3 < C)
            def _():
                issue(j + 3)

            b = lax.rem(j, 4)
            pltpu.make_async_copy(ones_v, acc_o.at[idx_s.at[0]],
                                  dsem.at[b]).wait()
            pltpu.make_async_copy(ones_v, acc_i.at[idx_d.at[0]],
                                  dsem.at[b]).wait()
            return carry

        lax.fori_loop(0, C, body, 0)
        plsc.subcore_barrier()
        pltpu.sync_copy(acc_o.at[pl.ds(s * RPT, RPT)],
                        out_hbm.at[pl.ds(c * N_PAD + s * RPT, RPT)])
        pltpu.sync_copy(acc_i.at[pl.ds(s * RPT, RPT)],
                        out_hbm.at[pl.ds((2 + c) * N_PAD + s * RPT, RPT)])

    return k(src2, dst2, zeros1)


def _propagate(xs, src2, dst2, zrows, d):
    @functools.partial(
        pl.kernel,
        out_type=jax.ShapeDtypeStruct((2 * N_PAD, d), jnp.float32),
        mesh=_sc_mesh(),
        scratch_types=[
            pltpu.VMEM((2, BC, K), jnp.int32),
            pltpu.VMEM((2, BC, K), jnp.int32),
            pltpu.VMEM((NBUF, K, d), jnp.float32),
            pltpu.SemaphoreType.DMA((NBUF,)),
            pltpu.SemaphoreType.DMA((2,)),
            pltpu.VMEM_SHARED((N_PAD, d), jnp.float32),
        ],
    )
    def k(xs_hbm, src_hbm, dst_hbm, z_hbm, out_hbm, sidx, didx, buf, gsem,
          isem, acc):
        c = lax.axis_index("c")
        s = lax.axis_index("s")
        base = (c * NS + s) * C

        def load_idx_block(blk, slot):
            off = base + blk * BC
            pltpu.async_copy(src_hbm.at[pl.ds(off, BC)], sidx.at[slot],
                             isem.at[slot])
            pltpu.async_copy(dst_hbm.at[pl.ds(off, BC)], didx.at[slot],
                             isem.at[slot])

        def wait_idx_block(slot):
            pltpu.make_async_copy(src_hbm.at[pl.ds(base, BC)], sidx.at[slot],
                                  isem.at[slot]).wait()
            pltpu.make_async_copy(src_hbm.at[pl.ds(base, BC)], didx.at[slot],
                                  isem.at[slot]).wait()

        load_idx_block(0, 0)
        load_idx_block(1, 1)
        pltpu.sync_copy(z_hbm, acc.at[pl.ds(s * RPT, RPT)])
        wait_idx_block(0)
        pltpu.async_copy(xs_hbm.at[sidx.at[0, 0]], buf.at[0], gsem.at[0])
        plsc.subcore_barrier()

        def body(j, carry):
            blk = lax.div(j, BC)

            @pl.when(jnp.logical_and(lax.rem(j, BC) == 0,
                                     jnp.logical_and(blk >= 1,
                                                     blk + 1 < NBLK)))
            def _():
                load_idx_block(blk + 1, lax.rem(blk + 1, 2))

            jn = j + 1
            slot_n = lax.rem(lax.div(jn, BC), 2)

            @pl.when(jn < C)
            def _():
                @pl.when(lax.rem(jn, BC) == 0)
                def _():
                    wait_idx_block(slot_n)

                bn = lax.rem(jn, NBUF)
                pltpu.async_copy(xs_hbm.at[sidx.at[slot_n, lax.rem(jn, BC)]],
                                 buf.at[bn], gsem.at[bn])

            b = lax.rem(j, NBUF)
            slot = lax.rem(blk, 2)
            pltpu.make_async_copy(xs_hbm.at[pl.ds(0, K)], buf.at[b],
                                  gsem.at[b]).wait()
            pltpu.sync_copy(buf.at[b], acc.at[didx.at[slot, lax.rem(j, BC)]],
                            add=True)
            return carry

        lax.fori_loop(0, C, body, 0)
        plsc.subcore_barrier()
        pltpu.sync_copy(acc.at[pl.ds(s * RPT, RPT)],
                        out_hbm.at[pl.ds(c * N_PAD + s * RPT, RPT)])

    return k(xs, src2, dst2, zrows)


def _norms(dblk):
    ns = lax.rsqrt(jnp.maximum(dblk[:, 0] + dblk[:, 1], 1.0))
    nd = lax.rsqrt(jnp.maximum(dblk[:, 2] + dblk[:, 3], 1.0))
    return ns, nd


def _tc_prelude(feats_p, degs_t):
    def body(f_ref, d_ref, o_ref):
        ns, _ = _norms(d_ref[...])
        o_ref[...] = f_ref[...] * ns[:, None]

    return pl.pallas_call(
        body,
        grid=(N_PAD // BR,),
        in_specs=[pl.BlockSpec((BR, D_HID), lambda i: (i, 0)),
                  pl.BlockSpec((BR, 4), lambda i: (i, 0))],
        out_specs=pl.BlockSpec((BR, D_HID), lambda i: (i, 0)),
        out_shape=jax.ShapeDtypeStruct((N_PAD, D_HID), jnp.float32),
    )(feats_p, degs_t)


def _tc_layer(parts, degs_t, W, b2d):
    def body(a0, a1, d_ref, w_ref, b_ref, o_ref):
        ns, nd = _norms(d_ref[...])
        agg = (a0[...] + a1[...]) * nd[:, None]
        h = jnp.dot(agg, w_ref[...], preferred_element_type=jnp.float32)
        h = jnp.maximum(h + b_ref[...], 0.0)
        o_ref[...] = h * ns[:, None]

    nblk = N_PAD // BR
    return pl.pallas_call(
        body,
        grid=(nblk,),
        in_specs=[pl.BlockSpec((BR, D_HID), lambda i: (i, 0)),
                  pl.BlockSpec((BR, D_HID), lambda i: (i + nblk, 0)),
                  pl.BlockSpec((BR, 4), lambda i: (i, 0)),
                  pl.BlockSpec((D_HID, D_HID), lambda i: (0, 0)),
                  pl.BlockSpec((1, D_HID), lambda i: (0, 0))],
        out_specs=pl.BlockSpec((BR, D_HID), lambda i: (i, 0)),
        out_shape=jax.ShapeDtypeStruct((N_PAD, D_HID), jnp.float32),
    )(parts, parts, degs_t, W, b2d)


def _tc_layer_prior(parts, degs_t, W, b2d):
    def body(a0, a1, d_ref, w_ref, b_ref, prior_ref, xs_ref):
        ns, nd = _norms(d_ref[...])
        agg = (a0[...] + a1[...]) * nd[:, None]
        h = jnp.dot(agg, w_ref[...], preferred_element_type=jnp.float32)
        h = jnp.maximum(h + b_ref[...], 0.0)
        prior_ref[...] = h
        xs_ref[...] = h * ns[:, None]

    nblk = N_PAD // BR
    return pl.pallas_call(
        body,
        grid=(nblk,),
        in_specs=[pl.BlockSpec((BR, D_HID), lambda i: (i, 0)),
                  pl.BlockSpec((BR, D_HID), lambda i: (i + nblk, 0)),
                  pl.BlockSpec((BR, 4), lambda i: (i, 0)),
                  pl.BlockSpec((D_HID, D_HID), lambda i: (0, 0)),
                  pl.BlockSpec((1, D_HID), lambda i: (0, 0))],
        out_specs=[pl.BlockSpec((BR, D_HID), lambda i: (i, 0)),
                   pl.BlockSpec((BR, D_HID), lambda i: (i, 0))],
        out_shape=[jax.ShapeDtypeStruct((N_PAD, D_HID), jnp.float32),
                   jax.ShapeDtypeStruct((N_PAD, D_HID), jnp.float32)],
    )(parts, parts, degs_t, W, b2d)


def _tc_final(parts, degs_t, W2p, b2d):
    def body(a0, a1, d_ref, w2_ref, b_ref, o_ref):
        _, nd = _norms(d_ref[...])
        agg = (a0[...] + a1[...]) * nd[:, None]
        h = jnp.dot(agg, w2_ref[...], preferred_element_type=jnp.float32)
        o_ref[...] = h + b_ref[...]

    nblk = N_PAD // BR
    return pl.pallas_call(
        body,
        grid=(nblk,),
        in_specs=[pl.BlockSpec((BR, D_HID), lambda i: (i, 0)),
                  pl.BlockSpec((BR, D_HID), lambda i: (i + nblk, 0)),
                  pl.BlockSpec((BR, 4), lambda i: (i, 0)),
                  pl.BlockSpec((D_HID, D_OUTP), lambda i: (0, 0)),
                  pl.BlockSpec((1, D_OUTP), lambda i: (0, 0))],
        out_specs=pl.BlockSpec((BR, D_OUTP), lambda i: (i, 0)),
        out_shape=jax.ShapeDtypeStruct((N_PAD, D_OUTP), jnp.float32),
    )(parts, parts, degs_t, W2p, b2d)


def kernel(features, edge_index, W0, b0, W1, b1, W2, b2):
    src = edge_index[0]
    dst = edge_index[1]
    pad = N_NODES + jnp.arange(E_PAD - E, dtype=jnp.int32) % (N_PAD - N_NODES)
    src2 = jnp.concatenate([src, pad]).reshape(NW * C, K)
    dst2 = jnp.concatenate([dst, pad]).reshape(NW * C, K)

    feats_p = jnp.zeros((N_PAD, D_HID), jnp.float32).at[:N_NODES].set(features)
    zeros1 = jnp.zeros((RPT,), jnp.float32)
    zrows_h = jnp.zeros((RPT, D_HID), jnp.float32)
    W2p = jnp.zeros((D_HID, D_OUTP), jnp.float32).at[:, :D_OUT].set(W2)
    b0d = jnp.reshape(b0, (1, D_HID))
    b1d = jnp.reshape(b1, (1, D_HID))
    b2d = jnp.zeros((1, D_OUTP), jnp.float32).at[0, :D_OUT].set(b2)

    degs = _degrees(src2, dst2, zeros1)
    degs_t = degs.reshape(4, N_PAD).T

    xs0 = _tc_prelude(feats_p, degs_t)
    parts = _propagate(xs0, src2, dst2, zrows_h, D_HID)
    xs1 = _tc_layer(parts, degs_t, W0, b0d)
    parts = _propagate(xs1, src2, dst2, zrows_h, D_HID)
    xs2 = _tc_layer(parts, degs_t, W1, b1d)
    parts = _propagate(xs2, src2, dst2, zrows_h, D_HID)
    prior_p, xs3 = _tc_layer_prior(parts, degs_t, W1, b1d)
    parts = _propagate(xs3, src2, dst2, zrows_h, D_HID)
    out_p = _tc_final(parts, degs_t, W2p, b2d)

    return out_p[:N_NODES, :D_OUT], prior_p[:N_NODES]


# speedup vs baseline: 1.4590x; 1.0591x over previous
import functools

import jax
import jax.numpy as jnp
from jax import lax
from jax.experimental import pallas as pl
from jax.experimental.pallas import tpu as pltpu
from jax.experimental.pallas import tpu_sc as plsc

N_NODES = 10000
D_HID = 128
D_OUT = 40
D_OUTP = 48
NC, NS = 2, 16
NW = NC * NS
K = 128
N_PAD = 10240
RPT = N_PAD // NS
E = 320000
C = 80
E_PAD = NW * C * K
BR = 5120
NBUF = 2
BC = 16
NBLK = C // BC


def _sc_mesh():
    return plsc.VectorSubcoreMesh(core_axis_name="c", subcore_axis_name="s")


def _degrees(src2, dst2, zeros1):
    @functools.partial(
        pl.kernel,
        out_type=jax.ShapeDtypeStruct((4 * N_PAD,), jnp.float32),
        mesh=_sc_mesh(),
        scratch_types=[
            pltpu.VMEM((C, K), jnp.int32),
            pltpu.VMEM((C, K), jnp.int32),
            pltpu.VMEM((K,), jnp.float32),
            pltpu.SemaphoreType.DMA((4,)),
            pltpu.VMEM_SHARED((N_PAD,), jnp.float32),
            pltpu.VMEM_SHARED((N_PAD,), jnp.float32),
        ],
    )
    def k(src_hbm, dst_hbm, z_hbm, out_hbm, idx_s, idx_d, ones_v, dsem,
          acc_o, acc_i):
        c = lax.axis_index("c")
        s = lax.axis_index("s")
        t = c * NS + s
        pltpu.sync_copy(src_hbm.at[pl.ds(t * C, C)], idx_s)
        pltpu.sync_copy(dst_hbm.at[pl.ds(t * C, C)], idx_d)
        for i in range(K // 16):
            ones_v[pl.ds(i * 16, 16)] = jnp.ones((16,), jnp.float32)
        pltpu.sync_copy(z_hbm, acc_o.at[pl.ds(s * RPT, RPT)])
        pltpu.sync_copy(z_hbm, acc_i.at[pl.ds(s * RPT, RPT)])
        plsc.subcore_barrier()

        def issue(j):
            b = lax.rem(j, 4)
            pltpu.async_copy(ones_v, acc_o.at[idx_s.at[j]], dsem.at[b],
                             add=True)
            pltpu.async_copy(ones_v, acc_i.at[idx_d.at[j]], dsem.at[b],
                             add=True)

        for j in range(3):
            issue(j)

        def body(j, carry):
            @pl.when(j + 3 < C)
            def _():
                issue(j + 3)

            b = lax.rem(j, 4)
            pltpu.make_async_copy(ones_v, acc_o.at[idx_s.at[0]],
                                  dsem.at[b]).wait()
            pltpu.make_async_copy(ones_v, acc_i.at[idx_d.at[0]],
                                  dsem.at[b]).wait()
            return carry

        lax.fori_loop(0, C, body, 0)
        plsc.subcore_barrier()
        pltpu.sync_copy(acc_o.at[pl.ds(s * RPT, RPT)],
                        out_hbm.at[pl.ds(c * N_PAD + s * RPT, RPT)])
        pltpu.sync_copy(acc_i.at[pl.ds(s * RPT, RPT)],
                        out_hbm.at[pl.ds((2 + c) * N_PAD + s * RPT, RPT)])

    return k(src2, dst2, zeros1)


def _propagate(xs, src2, dst2, zrows, d, sc_tiling=False):
    extra = {}
    if sc_tiling:
        extra["compiler_params"] = pltpu.CompilerParams(
            use_tc_tiling_on_sc=False)

    @functools.partial(
        pl.kernel,
        out_type=jax.ShapeDtypeStruct((2 * N_PAD, d), jnp.float32),
        mesh=_sc_mesh(),
        **extra,
        scratch_types=[
            pltpu.VMEM((2, BC, K), jnp.int32),
            pltpu.VMEM((2, BC, K), jnp.int32),
            pltpu.VMEM((NBUF, K, d), jnp.float32),
            pltpu.SemaphoreType.DMA((NBUF,)),
            pltpu.SemaphoreType.DMA((2,)),
            pltpu.VMEM_SHARED((N_PAD, d), jnp.float32),
        ],
    )
    def k(xs_hbm, src_hbm, dst_hbm, z_hbm, out_hbm, sidx, didx, buf, gsem,
          isem, acc):
        c = lax.axis_index("c")
        s = lax.axis_index("s")
        base = (c * NS + s) * C

        def load_idx_block(blk, slot):
            off = base + blk * BC
            pltpu.async_copy(src_hbm.at[pl.ds(off, BC)], sidx.at[slot],
                             isem.at[slot])
            pltpu.async_copy(dst_hbm.at[pl.ds(off, BC)], didx.at[slot],
                             isem.at[slot])

        def wait_idx_block(slot):
            pltpu.make_async_copy(src_hbm.at[pl.ds(base, BC)], sidx.at[slot],
                                  isem.at[slot]).wait()
            pltpu.make_async_copy(src_hbm.at[pl.ds(base, BC)], didx.at[slot],
                                  isem.at[slot]).wait()

        load_idx_block(0, 0)
        load_idx_block(1, 1)
        pltpu.sync_copy(z_hbm, acc.at[pl.ds(s * RPT, RPT)])
        wait_idx_block(0)
        pltpu.async_copy(xs_hbm.at[sidx.at[0, 0]], buf.at[0], gsem.at[0])
        plsc.subcore_barrier()

        def body(j, carry):
            blk = lax.div(j, BC)

            @pl.when(jnp.logical_and(lax.rem(j, BC) == 0,
                                     jnp.logical_and(blk >= 1,
                                                     blk + 1 < NBLK)))
            def _():
                load_idx_block(blk + 1, lax.rem(blk + 1, 2))

            jn = j + 1
            slot_n = lax.rem(lax.div(jn, BC), 2)

            @pl.when(jn < C)
            def _():
                @pl.when(lax.rem(jn, BC) == 0)
                def _():
                    wait_idx_block(slot_n)

                bn = lax.rem(jn, NBUF)
                pltpu.async_copy(xs_hbm.at[sidx.at[slot_n, lax.rem(jn, BC)]],
                                 buf.at[bn], gsem.at[bn])

            b = lax.rem(j, NBUF)
            slot = lax.rem(blk, 2)
            pltpu.make_async_copy(xs_hbm.at[pl.ds(0, K)], buf.at[b],
                                  gsem.at[b]).wait()
            pltpu.sync_copy(buf.at[b], acc.at[didx.at[slot, lax.rem(j, BC)]],
                            add=True)
            return carry

        lax.fori_loop(0, C, body, 0)
        plsc.subcore_barrier()
        pltpu.sync_copy(acc.at[pl.ds(s * RPT, RPT)],
                        out_hbm.at[pl.ds(c * N_PAD + s * RPT, RPT)])

    return k(xs, src2, dst2, zrows)


def _norms(dblk):
    ns = lax.rsqrt(jnp.maximum(dblk[:, 0] + dblk[:, 1], 1.0))
    nd = lax.rsqrt(jnp.maximum(dblk[:, 2] + dblk[:, 3], 1.0))
    return ns, nd


def _tc_prelude(feats_p, degs_t):
    def body(f_ref, d_ref, o_ref):
        ns, _ = _norms(d_ref[...])
        o_ref[...] = f_ref[...] * ns[:, None]

    return pl.pallas_call(
        body,
        grid=(N_PAD // BR,),
        in_specs=[pl.BlockSpec((BR, D_HID), lambda i: (i, 0)),
                  pl.BlockSpec((BR, 4), lambda i: (i, 0))],
        out_specs=pl.BlockSpec((BR, D_HID), lambda i: (i, 0)),
        out_shape=jax.ShapeDtypeStruct((N_PAD, D_HID), jnp.float32),
    )(feats_p, degs_t)


def _tc_layer(parts, degs_t, W, b2d):
    def body(a0, a1, d_ref, w_ref, b_ref, o_ref):
        ns, nd = _norms(d_ref[...])
        agg = (a0[...] + a1[...]) * nd[:, None]
        h = jnp.dot(agg, w_ref[...], preferred_element_type=jnp.float32)
        h = jnp.maximum(h + b_ref[...], 0.0)
        o_ref[...] = h * ns[:, None]

    nblk = N_PAD // BR
    return pl.pallas_call(
        body,
        grid=(nblk,),
        in_specs=[pl.BlockSpec((BR, D_HID), lambda i: (i, 0)),
                  pl.BlockSpec((BR, D_HID), lambda i: (i + nblk, 0)),
                  pl.BlockSpec((BR, 4), lambda i: (i, 0)),
                  pl.BlockSpec((D_HID, D_HID), lambda i: (0, 0)),
                  pl.BlockSpec((1, D_HID), lambda i: (0, 0))],
        out_specs=pl.BlockSpec((BR, D_HID), lambda i: (i, 0)),
        out_shape=jax.ShapeDtypeStruct((N_PAD, D_HID), jnp.float32),
    )(parts, parts, degs_t, W, b2d)


def _tc_layer_prior(parts, degs_t, W, b2d, W2p):
    def body(a0, a1, d_ref, w_ref, b_ref, w2_ref, prior_ref, g_ref):
        ns, nd = _norms(d_ref[...])
        agg = (a0[...] + a1[...]) * nd[:, None]
        h = jnp.dot(agg, w_ref[...], preferred_element_type=jnp.float32)
        h = jnp.maximum(h + b_ref[...], 0.0)
        prior_ref[...] = h
        g_ref[...] = jnp.dot(h * ns[:, None], w2_ref[...],
                             preferred_element_type=jnp.float32)

    nblk = N_PAD // BR
    return pl.pallas_call(
        body,
        grid=(nblk,),
        in_specs=[pl.BlockSpec((BR, D_HID), lambda i: (i, 0)),
                  pl.BlockSpec((BR, D_HID), lambda i: (i + nblk, 0)),
                  pl.BlockSpec((BR, 4), lambda i: (i, 0)),
                  pl.BlockSpec((D_HID, D_HID), lambda i: (0, 0)),
                  pl.BlockSpec((1, D_HID), lambda i: (0, 0)),
                  pl.BlockSpec((D_HID, D_OUTP), lambda i: (0, 0))],
        out_specs=[pl.BlockSpec((BR, D_HID), lambda i: (i, 0)),
                   pl.BlockSpec((BR, D_OUTP), lambda i: (i, 0))],
        out_shape=[jax.ShapeDtypeStruct((N_PAD, D_HID), jnp.float32),
                   jax.ShapeDtypeStruct((N_PAD, D_OUTP), jnp.float32)],
    )(parts, parts, degs_t, W, b2d, W2p)


def _tc_final(parts, degs_t, b2d):
    def body(a0, a1, d_ref, b_ref, o_ref):
        _, nd = _norms(d_ref[...])
        o_ref[...] = (a0[...] + a1[...]) * nd[:, None] + b_ref[...]

    nblk = N_PAD // BR
    return pl.pallas_call(
        body,
        grid=(nblk,),
        in_specs=[pl.BlockSpec((BR, D_OUTP), lambda i: (i, 0)),
                  pl.BlockSpec((BR, D_OUTP), lambda i: (i + nblk, 0)),
                  pl.BlockSpec((BR, 4), lambda i: (i, 0)),
                  pl.BlockSpec((1, D_OUTP), lambda i: (0, 0))],
        out_specs=pl.BlockSpec((BR, D_OUTP), lambda i: (i, 0)),
        out_shape=jax.ShapeDtypeStruct((N_PAD, D_OUTP), jnp.float32),
    )(parts, parts, degs_t, b2d)


def kernel(features, edge_index, W0, b0, W1, b1, W2, b2):
    src = edge_index[0]
    dst = edge_index[1]
    pad = N_NODES + jnp.arange(E_PAD - E, dtype=jnp.int32) % (N_PAD - N_NODES)
    src2 = jnp.concatenate([src, pad]).reshape(NW * C, K)
    dst2 = jnp.concatenate([dst, pad]).reshape(NW * C, K)

    feats_p = jnp.zeros((N_PAD, D_HID), jnp.float32).at[:N_NODES].set(features)
    zeros1 = jnp.zeros((RPT,), jnp.float32)
    zrows_h = jnp.zeros((RPT, D_HID), jnp.float32)
    zrows_o = jnp.zeros((RPT, D_OUTP), jnp.float32)
    W2p = jnp.zeros((D_HID, D_OUTP), jnp.float32).at[:, :D_OUT].set(W2)
    b0d = jnp.reshape(b0, (1, D_HID))
    b1d = jnp.reshape(b1, (1, D_HID))
    b2d = jnp.zeros((1, D_OUTP), jnp.float32).at[0, :D_OUT].set(b2)

    degs = _degrees(src2, dst2, zeros1)
    degs_t = degs.reshape(4, N_PAD).T

    xs0 = _tc_prelude(feats_p, degs_t)
    parts = _propagate(xs0, src2, dst2, zrows_h, D_HID)
    xs1 = _tc_layer(parts, degs_t, W0, b0d)
    parts = _propagate(xs1, src2, dst2, zrows_h, D_HID)
    xs2 = _tc_layer(parts, degs_t, W1, b1d)
    parts = _propagate(xs2, src2, dst2, zrows_h, D_HID)
    prior_p, g = _tc_layer_prior(parts, degs_t, W1, b1d, W2p)
    parts = _propagate(g, src2, dst2, zrows_o, D_OUTP, sc_tiling=True)
    out_p = _tc_final(parts, degs_t, b2d)

    return out_p[:N_NODES, :D_OUT], prior_p[:N_NODES]
